# initial kernel scaffold (unmeasured)
import jax
import jax.numpy as jnp
from jax import lax
from jax.experimental import pallas as pl
from jax.experimental.pallas import tpu as pltpu

B, H, D, BS = 16, 16, 64, 16
P_LOCAL = 128
NKEYS = P_LOCAL * BS
NBT = 128
NEG = -1e30


def _body(q_ref, k_ref, v_ref, bt_ref, lens_ref, out_ref,
          o_send, o_recv, ml_send, ml_recv, send_sems, recv_sems):
    my_x = lax.axis_index("x")
    my_y = lax.axis_index("y")
    my_z = lax.axis_index("z")
    peer = (my_x, 1 - my_y, my_z)

    barrier = pltpu.get_barrier_semaphore()
    pl.semaphore_signal(barrier, inc=1, device_id=peer,
                        device_id_type=pl.DeviceIdType.MESH)
    pl.semaphore_wait(barrier, 1)

    bt = bt_ref[:, :]
    lens_c = lens_ref[:, :]
    j_idx = lax.broadcasted_iota(jnp.int32, (B, NBT), 1)
    valid = j_idx < lens_c
    page3 = (lax.broadcasted_iota(jnp.int32, (B, NBT, P_LOCAL), 2)
             + my_y * P_LOCAL)
    match = (bt[:, :, None] == page3) & valid[:, :, None]
    w = jnp.sum(match.astype(jnp.float32), axis=1)

    p_io = lax.broadcasted_iota(jnp.int32, (P_LOCAL, NKEYS), 0)
    k_io = lax.broadcasted_iota(jnp.int32, (P_LOCAL, NKEYS), 1) // BS
    expand = jnp.where(p_io == k_io, 1.0, 0.0).astype(jnp.float32)
    w_keys = jnp.dot(w, expand, preferred_element_type=jnp.float32)
    key_live = w_keys > 0.0

    q3 = q_ref[:, 0, :, :]
    k3 = k_ref[:, :, :, :].reshape(NKEYS, H, D)
    v3 = v_ref[:, :, :, :].reshape(NKEYS, H, D)

    scale = D ** -0.5
    m_parts, l_parts, o_parts = [], [], []
    for h in range(H):
        q_h = q3[:, h, :]
        k_h = k3[:, h, :]
        v_h = v3[:, h, :]
        s = lax.dot_general(q_h, k_h, (((1,), (1,)), ((), ())),
                            preferred_element_type=jnp.float32) * scale
        s = jnp.where(key_live, s, NEG)
        m_h = jnp.max(s, axis=1, keepdims=True)
        p = jnp.exp(s - m_h) * w_keys
        l_h = jnp.sum(p, axis=1, keepdims=True)
        o_h = jnp.dot(p, v_h, preferred_element_type=jnp.float32)
        m_parts.append(m_h)
        l_parts.append(l_h)
        o_parts.append(o_h.reshape(1, B, D))

    m_loc = jnp.concatenate(m_parts, axis=1)
    l_loc = jnp.concatenate(l_parts, axis=1)
    o_loc = jnp.concatenate(o_parts, axis=0)

    o_send[:, :, :] = o_loc
    ml_send[0, :, :] = m_loc
    ml_send[1, :, :] = l_loc

    rdma_o = pltpu.make_async_remote_copy(
        src_ref=o_send, dst_ref=o_recv,
        send_sem=send_sems.at[0], recv_sem=recv_sems.at[0],
        device_id=peer, device_id_type=pl.DeviceIdType.MESH)
    rdma_ml = pltpu.make_async_remote_copy(
        src_ref=ml_send, dst_ref=ml_recv,
        send_sem=send_sems.at[1], recv_sem=recv_sems.at[1],
        device_id=peer, device_id_type=pl.DeviceIdType.MESH)
    rdma_o.start()
    rdma_ml.start()
    rdma_o.wait_recv()
    rdma_ml.wait_recv()

    m_peer = ml_recv[0, :, :]
    l_peer = ml_recv[1, :, :]
    o_peer = o_recv[:, :, :]

    m_star = jnp.maximum(m_loc, m_peer)
    a_loc = jnp.exp(m_loc - m_star)
    a_peer = jnp.exp(m_peer - m_star)
    l_tot = l_loc * a_loc + l_peer * a_peer

    o_l = jnp.transpose(o_loc, (1, 0, 2))
    o_p = jnp.transpose(o_peer, (1, 0, 2))
    out = (o_l * a_loc[:, :, None] + o_p * a_peer[:, :, None]) / l_tot[:, :, None]
    out_ref[:, 0, :, :] = out

    rdma_o.wait_send()
    rdma_ml.wait_send()


def kernel(Q, K, V, bt, lens):
    lens2 = lens.reshape(B, 1)
    return pl.pallas_call(
        _body,
        out_shape=jax.ShapeDtypeStruct((B, 1, H, D), jnp.float32),
        in_specs=[pl.BlockSpec(memory_space=pltpu.VMEM)] * 5,
        out_specs=pl.BlockSpec(memory_space=pltpu.VMEM),
        scratch_shapes=[
            pltpu.VMEM((H, B, D), jnp.float32),
            pltpu.VMEM((H, B, D), jnp.float32),
            pltpu.VMEM((2, B, H), jnp.float32),
            pltpu.VMEM((2, B, H), jnp.float32),
            pltpu.SemaphoreType.DMA((2,)),
            pltpu.SemaphoreType.DMA((2,)),
        ],
        compiler_params=pltpu.CompilerParams(collective_id=0),
    )(Q, K, V, bt, lens2)


# baseline (device time: 62437 ns/iter reference)
import jax
import jax.numpy as jnp
from jax import lax
from jax.experimental import pallas as pl
from jax.experimental.pallas import tpu as pltpu

B, H, D, BS = 16, 16, 64, 16
P_LOCAL = 128
NKEYS = P_LOCAL * BS
NBT = 128
NEG = -1e30


def _body(q_ref, k_ref, v_ref, bt_ref, out_ref,
          o_send, o_recv, ml_send, ml_recv, send_sems, recv_sems):
    my_x = lax.axis_index("x")
    my_y = lax.axis_index("y")
    my_z = lax.axis_index("z")
    peer = (my_x, 1 - my_y, my_z)

    barrier = pltpu.get_barrier_semaphore()
    pl.semaphore_signal(barrier, inc=1, device_id=peer,
                        device_id_type=pl.DeviceIdType.MESH)
    pl.semaphore_wait(barrier, 1)

    bt3 = bt_ref[:, :, :]
    page3 = (lax.broadcasted_iota(jnp.int32, (B, NBT, P_LOCAL), 2)
             + my_y * P_LOCAL)
    match = (bt3 == page3).astype(jnp.float32)
    w = jnp.sum(match, axis=1)

    p_io = lax.broadcasted_iota(jnp.int32, (P_LOCAL, NKEYS), 0)
    k_io = lax.broadcasted_iota(jnp.int32, (P_LOCAL, NKEYS), 1) // BS
    expand = jnp.where(p_io == k_io, 1.0, 0.0).astype(jnp.float32)
    w_keys = jnp.dot(w, expand, preferred_element_type=jnp.float32)
    key_live = w_keys > 0.0

    q3 = q_ref[:, 0, :, :]
    k3 = k_ref[:, :, :, :].reshape(NKEYS, H, D)
    v3 = v_ref[:, :, :, :].reshape(NKEYS, H, D)

    scale = D ** -0.5
    for h in range(H):
        q_h = q3[:, h, :]
        k_h = k3[:, h, :]
        v_h = v3[:, h, :]
        s = lax.dot_general(q_h, k_h, (((1,), (1,)), ((), ())),
                            preferred_element_type=jnp.float32) * scale
        s = jnp.where(key_live, s, NEG)
        m_h = jnp.max(s, axis=1, keepdims=True)
        p = jnp.exp(s - m_h) * w_keys
        l_h = jnp.sum(p, axis=1, keepdims=True)
        o_h = jnp.dot(p, v_h, preferred_element_type=jnp.float32)
        o_send[h, :, :] = o_h
        ml_send[0, :, h:h + 1] = m_h
        ml_send[1, :, h:h + 1] = l_h

    rdma_o = pltpu.make_async_remote_copy(
        src_ref=o_send, dst_ref=o_recv,
        send_sem=send_sems.at[0], recv_sem=recv_sems.at[0],
        device_id=peer, device_id_type=pl.DeviceIdType.MESH)
    rdma_ml = pltpu.make_async_remote_copy(
        src_ref=ml_send, dst_ref=ml_recv,
        send_sem=send_sems.at[1], recv_sem=recv_sems.at[1],
        device_id=peer, device_id_type=pl.DeviceIdType.MESH)
    rdma_o.start()
    rdma_ml.start()
    rdma_o.wait_recv()
    rdma_ml.wait_recv()

    m_loc = ml_send[0, :, :]
    l_loc = ml_send[1, :, :]
    m_peer = ml_recv[0, :, :]
    l_peer = ml_recv[1, :, :]

    m_star = jnp.maximum(m_loc, m_peer)
    a_loc = jnp.exp(m_loc - m_star)
    a_peer = jnp.exp(m_peer - m_star)
    l_tot = l_loc * a_loc + l_peer * a_peer

    for h in range(H):
        num = (o_send[h, :, :] * a_loc[:, h:h + 1]
               + o_recv[h, :, :] * a_peer[:, h:h + 1])
        out_ref[:, 0, h, :] = num / l_tot[:, h:h + 1]

    rdma_o.wait_send()
    rdma_ml.wait_send()


def kernel(Q, K, V, bt, lens):
    j = jnp.arange(NBT, dtype=jnp.int32)[None, :]
    bt_eff = jnp.where(j < lens[:, None], bt, -1).reshape(B, NBT, 1)
    return pl.pallas_call(
        _body,
        out_shape=jax.ShapeDtypeStruct((B, 1, H, D), jnp.float32),
        in_specs=[pl.BlockSpec(memory_space=pltpu.VMEM)] * 4,
        out_specs=pl.BlockSpec(memory_space=pltpu.VMEM),
        scratch_shapes=[
            pltpu.VMEM((H, B, D), jnp.float32),
            pltpu.VMEM((H, B, D), jnp.float32),
            pltpu.VMEM((2, B, H), jnp.float32),
            pltpu.VMEM((2, B, H), jnp.float32),
            pltpu.SemaphoreType.DMA((2,)),
            pltpu.SemaphoreType.DMA((2,)),
        ],
        compiler_params=pltpu.CompilerParams(collective_id=0),
    )(Q, K, V, bt_eff)


# device time: 24071 ns/iter; 2.5939x vs baseline; 2.5939x over previous
import jax
import jax.numpy as jnp
from jax import lax
from jax.experimental import pallas as pl
from jax.experimental.pallas import tpu as pltpu

B, H, D, BS = 16, 16, 64, 16
P_LOCAL = 128
NBT = 128
NEG = -1e30


def _body(q_ref, k_ref, v_ref, bt_ref, out_ref,
          o_send, o_recv, ml_send, ml_recv, send_sems, recv_sems):
    my_x = lax.axis_index("x")
    my_y = lax.axis_index("y")
    my_z = lax.axis_index("z")
    peer = (my_x, 1 - my_y, my_z)

    barrier = pltpu.get_barrier_semaphore()
    pl.semaphore_signal(barrier, inc=1, device_id=peer,
                        device_id_type=pl.DeviceIdType.MESH)
    pl.semaphore_wait(barrier, 1)

    bt3 = bt_ref[:, :, :]
    page3 = (lax.broadcasted_iota(jnp.int32, (B, NBT, P_LOCAL), 2)
             + my_y * P_LOCAL)
    match = (bt3 == page3).astype(jnp.float32)
    w = jnp.sum(match, axis=1)
    w3 = jnp.broadcast_to(w.reshape(1, B, P_LOCAL), (BS, B, P_LOCAL))
    live3 = w3 > 0.0

    q3 = q_ref[:, 0, :, :]

    scale = D ** -0.5
    for h in range(H):
        q_h = q3[:, h, :]
        k_h = k_ref[:, h, :, :]
        v_h = v_ref[:, h, :, :]
        q_b = jnp.broadcast_to(q_h.reshape(1, B, D), (BS, B, D))
        s = lax.dot_general(q_b, k_h, (((2,), (1,)), ((0,), (0,))),
                            preferred_element_type=jnp.float32) * scale
        s = jnp.where(live3, s, NEG)
        m3 = jnp.max(s, axis=(0, 2), keepdims=True)
        p = jnp.exp(s - m3) * w3
        l3 = jnp.sum(p, axis=(0, 2), keepdims=True)
        o3 = lax.dot_general(p, v_h, (((2,), (2,)), ((0,), (0,))),
                             preferred_element_type=jnp.float32)
        o_h = jnp.sum(o3, axis=0)
        o_send[h, :, :] = o_h
        ml_send[0, :, h:h + 1] = m3[0]
        ml_send[1, :, h:h + 1] = l3[0]

    rdma_o = pltpu.make_async_remote_copy(
        src_ref=o_send, dst_ref=o_recv,
        send_sem=send_sems.at[0], recv_sem=recv_sems.at[0],
        device_id=peer, device_id_type=pl.DeviceIdType.MESH)
    rdma_ml = pltpu.make_async_remote_copy(
        src_ref=ml_send, dst_ref=ml_recv,
        send_sem=send_sems.at[1], recv_sem=recv_sems.at[1],
        device_id=peer, device_id_type=pl.DeviceIdType.MESH)
    rdma_o.start()
    rdma_ml.start()
    rdma_o.wait_recv()
    rdma_ml.wait_recv()

    m_loc = ml_send[0, :, :]
    l_loc = ml_send[1, :, :]
    m_peer = ml_recv[0, :, :]
    l_peer = ml_recv[1, :, :]

    m_star = jnp.maximum(m_loc, m_peer)
    a_loc = jnp.exp(m_loc - m_star)
    a_peer = jnp.exp(m_peer - m_star)
    l_tot = l_loc * a_loc + l_peer * a_peer

    for h in range(H):
        num = (o_send[h, :, :] * a_loc[:, h:h + 1]
               + o_recv[h, :, :] * a_peer[:, h:h + 1])
        out_ref[:, 0, h, :] = num / l_tot[:, h:h + 1]

    rdma_o.wait_send()
    rdma_ml.wait_send()


def kernel(Q, K, V, bt, lens):
    j = jnp.arange(NBT, dtype=jnp.int32)[None, :]
    bt_eff = jnp.where(j < lens[:, None], bt, -1).reshape(B, NBT, 1)
    K_t = jnp.transpose(K, (1, 2, 3, 0))
    V_t = jnp.transpose(V, (1, 2, 3, 0))
    return pl.pallas_call(
        _body,
        out_shape=jax.ShapeDtypeStruct((B, 1, H, D), jnp.float32),
        in_specs=[pl.BlockSpec(memory_space=pltpu.VMEM)] * 4,
        out_specs=pl.BlockSpec(memory_space=pltpu.VMEM),
        scratch_shapes=[
            pltpu.VMEM((H, B, D), jnp.float32),
            pltpu.VMEM((H, B, D), jnp.float32),
            pltpu.VMEM((2, B, H), jnp.float32),
            pltpu.VMEM((2, B, H), jnp.float32),
            pltpu.SemaphoreType.DMA((2,)),
            pltpu.SemaphoreType.DMA((2,)),
        ],
        compiler_params=pltpu.CompilerParams(collective_id=0),
    )(Q, K_t, V_t, bt_eff)
